# baseline (speedup 1.0000x reference)
import functools

import jax
import jax.numpy as jnp
from jax import lax
from jax.experimental import pallas as pl
from jax.experimental.pallas import tpu as pltpu
from jax.experimental.pallas import tpu_sc as plsc

NUM_SCENES = 1000
D = 128
BATCH = 16384

_INFO = plsc.get_sparse_core_info()
_NC = _INFO.num_cores
_NS = _INFO.num_subcores

_A = 8192
_B_TC = BATCH - _A
_PER_S = _A // _NS
_B_C0 = ((_PER_S * 464) // 1024 // 8) * 8
_B_C1 = _PER_S - _B_C0

_TC_BLK = 1024


def _make_sc_gather():
    mesh = plsc.VectorSubcoreMesh(core_axis_name="c", subcore_axis_name="s")

    @functools.partial(
        pl.kernel,
        mesh=mesh,
        out_type=jax.ShapeDtypeStruct((BATCH, D), jnp.float32),
        scratch_types=[
            pltpu.VMEM((max(_B_C0, _B_C1),), jnp.int32),
            pltpu.VMEM((max(_B_C0, _B_C1), D), jnp.float32),
            pltpu.SemaphoreType.DMA,
        ],
    )
    def gather_kernel(idx_hbm, table_hbm, out_hbm, idx_v, rows_v, sem):
        c = lax.axis_index("c")
        s = lax.axis_index("s")

        def work(nb, base):
            pltpu.sync_copy(idx_hbm.at[pl.ds(base, nb)], idx_v.at[pl.ds(0, nb)])
            pltpu.async_copy(table_hbm.at[idx_v.at[pl.ds(0, nb)]],
                             rows_v.at[pl.ds(0, nb)], sem).wait()
            pltpu.sync_copy(rows_v.at[pl.ds(0, nb)],
                            out_hbm.at[pl.ds(base, nb)])

        @pl.when(c == 0)
        def _():
            work(_B_C0, s * _PER_S)

        @pl.when(c != 0)
        def _():
            work(_B_C1, s * _PER_S + _B_C0)

    return gather_kernel


_sc_gather = _make_sc_gather()


def _tc_body(idx_ref, table_ref, sc_ref, o_ref):
    del sc_ref
    ids = idx_ref[...]
    iota = lax.broadcasted_iota(jnp.int32, (_TC_BLK, NUM_SCENES), 1)
    onehot = (ids == iota).astype(jnp.bfloat16)
    tab = table_ref[...].astype(jnp.bfloat16)
    o_ref[...] = lax.dot_general(onehot, tab, (((1,), (0,)), ((), ())),
                                 preferred_element_type=jnp.float32)


_tc_fill = pl.pallas_call(
    _tc_body,
    grid=(_B_TC // _TC_BLK,),
    in_specs=[
        pl.BlockSpec((_TC_BLK, 1), lambda i: (i, 0)),
        pl.BlockSpec((NUM_SCENES, D), lambda i: (0, 0)),
        pl.BlockSpec(memory_space=pl.ANY),
    ],
    out_specs=pl.BlockSpec((_TC_BLK, D), lambda i: (i + _A // _TC_BLK, 0)),
    out_shape=jax.ShapeDtypeStruct((BATCH, D), jnp.float32),
    input_output_aliases={2: 0},
)


def kernel(scene_id, embedding_weight):
    if scene_id.ndim > 1:
        scene_id = jnp.squeeze(scene_id, axis=-1)
    scene_id = scene_id.astype(jnp.int32)
    sc_out = _sc_gather(scene_id, embedding_weight)
    idx_tc = scene_id[_A:].reshape(_B_TC, 1)
    return _tc_fill(idx_tc, embedding_weight, sc_out)


# speedup vs baseline: 1.1882x; 1.1882x over previous
import functools

import jax
import jax.numpy as jnp
from jax import lax
from jax.experimental import pallas as pl
from jax.experimental.pallas import tpu as pltpu
from jax.experimental.pallas import tpu_sc as plsc

NUM_SCENES = 1000
D = 128
BATCH = 16384

_INFO = plsc.get_sparse_core_info()
_NC = _INFO.num_cores
_NS = _INFO.num_subcores

_TC_BLK = 1024
_A = 10240
_B_TC = BATCH - _A
_PER_S = _A // _NS
_B_C0 = ((_PER_S * 464) // 1024 // 8) * 8
_B_C1 = _PER_S - _B_C0


def _make_sc_gather():
    mesh = plsc.VectorSubcoreMesh(core_axis_name="c", subcore_axis_name="s")

    @functools.partial(
        pl.kernel,
        mesh=mesh,
        out_type=(),
        scratch_types=[
            pltpu.VMEM((max(_B_C0, _B_C1),), jnp.int32),
            pltpu.VMEM((max(_B_C0, _B_C1), D), jnp.float32),
            pltpu.SemaphoreType.DMA,
        ],
    )
    def gather_kernel(idx_hbm, table_hbm, out_hbm, idx_v, rows_v, sem):
        c = lax.axis_index("c")
        s = lax.axis_index("s")

        def work(nb, base):
            pltpu.sync_copy(idx_hbm.at[pl.ds(base, nb)], idx_v.at[pl.ds(0, nb)])
            pltpu.async_copy(table_hbm.at[idx_v.at[pl.ds(0, nb)]],
                             rows_v.at[pl.ds(0, nb)], sem).wait()
            pltpu.sync_copy(rows_v.at[pl.ds(0, nb)],
                            out_hbm.at[pl.ds(base, nb)])

        @pl.when(c == 0)
        def _():
            work(_B_C0, s * _PER_S)

        @pl.when(c != 0)
        def _():
            work(_B_C1, s * _PER_S + _B_C0)

    return gather_kernel


_sc_gather = _make_sc_gather()


def _tc_body(idx_ref, table_ref, o_ref):
    ids = idx_ref[...]
    iota = lax.broadcasted_iota(jnp.int32, (NUM_SCENES, _TC_BLK), 0)
    onehot_t = (iota == ids).astype(jnp.bfloat16)
    tab = table_ref[...].astype(jnp.bfloat16)
    o_ref[...] = lax.dot_general(onehot_t, tab, (((0,), (0,)), ((), ())),
                                 preferred_element_type=jnp.float32)


_tc_partial = pl.pallas_call(
    _tc_body,
    grid=(_B_TC // _TC_BLK,),
    in_specs=[
        pl.BlockSpec((1, _TC_BLK), lambda i: (0, i + _A // _TC_BLK)),
        pl.BlockSpec((NUM_SCENES, D), lambda i: (0, 0)),
    ],
    out_specs=pl.BlockSpec((_TC_BLK, D), lambda i: (i + _A // _TC_BLK, 0)),
    out_shape=jax.ShapeDtypeStruct((BATCH, D), jnp.float32),
)


def kernel(scene_id, embedding_weight):
    if scene_id.ndim > 1:
        scene_id = jnp.squeeze(scene_id, axis=-1)
    scene_id = scene_id.astype(jnp.int32)
    tc_out = _tc_partial(scene_id.reshape(1, BATCH), embedding_weight)
    out_ref = jax.new_ref(tc_out)
    _sc_gather(scene_id, embedding_weight, out_ref)
    return out_ref[...]


# speedup vs baseline: 1.2326x; 1.0373x over previous
import functools

import jax
import jax.numpy as jnp
from jax import lax
from jax.experimental import pallas as pl
from jax.experimental.pallas import tpu as pltpu
from jax.experimental.pallas import tpu_sc as plsc

NUM_SCENES = 1000
D = 128
BATCH = 16384

_INFO = plsc.get_sparse_core_info()
_NC = _INFO.num_cores
_NS = _INFO.num_subcores
_PER_S = BATCH // _NS
_B_C0 = 464
_B_C1 = _PER_S - _B_C0


def _make_gather():
    mesh = plsc.VectorSubcoreMesh(core_axis_name="c", subcore_axis_name="s")

    @functools.partial(
        pl.kernel,
        mesh=mesh,
        out_type=jax.ShapeDtypeStruct((BATCH, D), jnp.float32),
        scratch_types=[
            pltpu.VMEM((max(_B_C0, _B_C1),), jnp.int32),
            pltpu.VMEM((max(_B_C0, _B_C1), D), jnp.float32),
            pltpu.SemaphoreType.DMA,
        ],
    )
    def gather_kernel(idx_hbm, table_hbm, out_hbm, idx_v, rows_v, sem):
        c = lax.axis_index("c")
        s = lax.axis_index("s")

        def work(nb, base):
            pltpu.sync_copy(idx_hbm.at[pl.ds(base, nb)], idx_v.at[pl.ds(0, nb)])
            pltpu.async_copy(table_hbm.at[idx_v.at[pl.ds(0, nb)]],
                             rows_v.at[pl.ds(0, nb)], sem).wait()
            pltpu.sync_copy(rows_v.at[pl.ds(0, nb)],
                            out_hbm.at[pl.ds(base, nb)])

        @pl.when(c == 0)
        def _():
            work(_B_C0, s * _PER_S)

        @pl.when(c != 0)
        def _():
            work(_B_C1, s * _PER_S + _B_C0)

    return gather_kernel


_gather = _make_gather()


def kernel(scene_id, embedding_weight):
    if scene_id.ndim > 1:
        scene_id = jnp.squeeze(scene_id, axis=-1)
    return _gather(scene_id.astype(jnp.int32), embedding_weight)
